# baseline (device time: 714273 ns/iter reference)
import jax
import jax.numpy as jnp
from jax import lax
from jax.experimental import pallas as pl
from jax.experimental.pallas import tpu as pltpu

N_DEV = 8
N_LOCAL_E = 8
CAP = 128
NSLOT = N_LOCAL_E * CAP


def _a2a_kernel(n_tok, d_model):
    def body(x_ref, tokc_ref, tokr_ref, g_ref, ew_ref,
             out_ref, xrecv_ref, ysend_ref, yret_ref,
             xsend_vmem, xa_vmem, w_vmem, ya_vmem, y_vmem,
             local_sem, xa_sem, w_sem, ya_sem, y_sem,
             send1, recv1, send2, recv2):
        my = lax.axis_index("i")

        xb = x_ref[:, :].astype(jnp.bfloat16)

        iota_d = lax.broadcasted_iota(jnp.int32, (NSLOT, n_tok), 1)
        disp = []
        local_cp = None
        for k in range(N_DEV):
            dest = lax.rem(my + k, N_DEV)
            tok_col = tokc_ref[pl.ds(dest * NSLOT, NSLOT), :]
            oh = (tok_col == iota_d).astype(jnp.bfloat16)
            xsend_vmem[k, :, :] = jnp.dot(
                oh, xb, preferred_element_type=jnp.float32
            ).astype(jnp.bfloat16)
            if k == 0:
                local_cp = pltpu.make_async_copy(
                    xsend_vmem.at[0], xrecv_ref.at[my], local_sem
                )
                local_cp.start()
            else:
                rdma = pltpu.make_async_remote_copy(
                    src_ref=xsend_vmem.at[k],
                    dst_ref=xrecv_ref.at[my],
                    send_sem=send1.at[k - 1],
                    recv_sem=recv1.at[k - 1],
                    device_id=(dest,),
                    device_id_type=pl.DeviceIdType.MESH,
                )
                rdma.start()
                disp.append(rdma)
        local_cp.wait()
        for rdma in disp:
            rdma.wait_recv()

        def expert_step(s, carry):
            cw = pltpu.make_async_copy(ew_ref.at[s], w_vmem, w_sem)
            cw.start()
            cx = pltpu.make_async_copy(
                xrecv_ref.at[:, pl.ds(s * CAP, CAP)], xa_vmem, xa_sem
            )
            cx.start()
            cx.wait()
            cw.wait()
            xa = xa_vmem[:, :, :].reshape(N_DEV * CAP, d_model)
            y = jnp.dot(xa, w_vmem[:, :], preferred_element_type=jnp.float32)
            ya_vmem[:, :, :] = y.astype(jnp.bfloat16).reshape(
                N_DEV, CAP, d_model
            )
            cy = pltpu.make_async_copy(
                ya_vmem, ysend_ref.at[:, pl.ds(s * CAP, CAP)], ya_sem
            )
            cy.start()
            cy.wait()
            return carry

        lax.fori_loop(0, N_LOCAL_E, expert_step, 0)

        ret = []
        for k in range(1, N_DEV):
            dest = lax.rem(my + k, N_DEV)
            rdma = pltpu.make_async_remote_copy(
                src_ref=ysend_ref.at[dest],
                dst_ref=yret_ref.at[my],
                send_sem=send2.at[k - 1],
                recv_sem=recv2.at[k - 1],
                device_id=(dest,),
                device_id_type=pl.DeviceIdType.MESH,
            )
            rdma.start()
            ret.append(rdma)
        local_cp = pltpu.make_async_copy(
            ysend_ref.at[my], yret_ref.at[my], local_sem
        )
        local_cp.start()

        out_ref[:, :] = jnp.zeros_like(out_ref)
        iota_c = lax.broadcasted_iota(jnp.int32, (n_tok, NSLOT), 0)
        for k in range(N_DEV):
            o = lax.rem(my - k + N_DEV, N_DEV)
            if k == 0:
                local_cp.wait()
            else:
                ret[k - 1].wait_recv()
            cy = pltpu.make_async_copy(yret_ref.at[o], y_vmem, y_sem)
            cy.start()
            cy.wait()
            tok_row = tokr_ref[:, pl.ds(o * NSLOT, NSLOT)]
            g_row = g_ref[:, pl.ds(o * NSLOT, NSLOT)]
            m = jnp.where(tok_row == iota_c, g_row, 0.0).astype(jnp.bfloat16)
            out_ref[:, :] += jnp.dot(
                m, y_vmem[:, :], preferred_element_type=jnp.float32
            )
        for rdma in disp:
            rdma.wait_send()
        for rdma in ret:
            rdma.wait_send()

    return pl.pallas_call(
        body,
        out_shape=(
            jax.ShapeDtypeStruct((n_tok, d_model), jnp.float32),
            jax.ShapeDtypeStruct((N_DEV, NSLOT, d_model), jnp.bfloat16),
            jax.ShapeDtypeStruct((N_DEV, NSLOT, d_model), jnp.bfloat16),
            jax.ShapeDtypeStruct((N_DEV, NSLOT, d_model), jnp.bfloat16),
        ),
        in_specs=[
            pl.BlockSpec(memory_space=pltpu.MemorySpace.VMEM),
            pl.BlockSpec(memory_space=pltpu.MemorySpace.VMEM),
            pl.BlockSpec(memory_space=pltpu.MemorySpace.VMEM),
            pl.BlockSpec(memory_space=pltpu.MemorySpace.VMEM),
            pl.BlockSpec(memory_space=pl.ANY),
        ],
        out_specs=(
            pl.BlockSpec(memory_space=pltpu.MemorySpace.VMEM),
            pl.BlockSpec(memory_space=pl.ANY),
            pl.BlockSpec(memory_space=pl.ANY),
            pl.BlockSpec(memory_space=pl.ANY),
        ),
        scratch_shapes=[
            pltpu.VMEM((N_DEV, NSLOT, d_model), jnp.bfloat16),
            pltpu.VMEM((N_DEV, CAP, d_model), jnp.bfloat16),
            pltpu.VMEM((d_model, d_model), jnp.bfloat16),
            pltpu.VMEM((N_DEV, CAP, d_model), jnp.bfloat16),
            pltpu.VMEM((NSLOT, d_model), jnp.bfloat16),
            pltpu.SemaphoreType.DMA,
            pltpu.SemaphoreType.DMA,
            pltpu.SemaphoreType.DMA,
            pltpu.SemaphoreType.DMA,
            pltpu.SemaphoreType.DMA,
            pltpu.SemaphoreType.DMA((N_DEV - 1,)),
            pltpu.SemaphoreType.DMA((N_DEV - 1,)),
            pltpu.SemaphoreType.DMA((N_DEV - 1,)),
            pltpu.SemaphoreType.DMA((N_DEV - 1,)),
        ],
        compiler_params=pltpu.CompilerParams(
            has_side_effects=True,
            vmem_limit_bytes=120 * 1024 * 1024,
        ),
    )


def kernel(x, router_W, route_idx, expert_W):
    n_tok, d_model = x.shape
    n_exp = router_W.shape[1]

    scores = jnp.dot(x, router_W, preferred_element_type=jnp.float32)
    p = jax.nn.softmax(scores, axis=-1)
    g = jnp.take_along_axis(p, route_idx, axis=1)
    g = g / jnp.sum(g, axis=1, keepdims=True)

    flat_e = route_idx.reshape(-1)
    flat_t = jnp.arange(2 * n_tok, dtype=jnp.int32) // 2
    flat_g = g.reshape(-1)
    order = jnp.argsort(flat_e, stable=True)
    se = flat_e[order]
    st = flat_t[order]
    sg = flat_g[order]
    start = jnp.searchsorted(se, jnp.arange(n_exp, dtype=se.dtype)).astype(
        jnp.int32
    )
    j = jnp.arange(n_exp * CAP, dtype=jnp.int32)
    e_of = j // CAP
    c_of = j % CAP
    pos = start[e_of] + c_of
    pc = jnp.minimum(pos, 2 * n_tok - 1)
    valid = (pos < 2 * n_tok) & (se[pc] == e_of)
    tok = jnp.where(valid, st[pc], n_tok)
    gsl = jnp.where(valid, sg[pc], 0.0)

    out, _, _, _ = _a2a_kernel(n_tok, d_model)(
        x,
        tok.reshape(-1, 1),
        tok.reshape(1, -1),
        gsl.reshape(1, -1),
        expert_W.astype(jnp.bfloat16),
    )
    return out


# device time: 391427 ns/iter; 1.8248x vs baseline; 1.8248x over previous
import jax
import jax.numpy as jnp
from jax import lax
from jax.experimental import pallas as pl
from jax.experimental.pallas import tpu as pltpu

N_DEV = 8
N_LOCAL_E = 8
CAP = 128
NSLOT = N_LOCAL_E * CAP


def _a2a_kernel(n_tok, d_model):
    def body(x_ref, s0r_ref, s1r_ref, s0c_ref, s1c_ref, g0c_ref, g1c_ref,
             ew_ref,
             out_ref, xrecv_ref, ysend_ref, yret_ref,
             xsend_vmem, xa_vmem, w_vmem, ya_vmem, y_vmem,
             local_sem, xa_sem, w_sem, ya_sem, y_sem,
             send1, recv1, send2, recv2):
        my = lax.axis_index("i")

        xb = x_ref[:, :].astype(jnp.bfloat16)

        iota_jc = lax.broadcasted_iota(jnp.int32, (NSLOT, n_tok), 0)
        disp = []
        local_cp = None
        for k in range(N_DEV):
            dest = lax.rem(my + k, N_DEV)
            jg = iota_jc + dest * NSLOT
            oh = ((s0r_ref[:, :] == jg) | (s1r_ref[:, :] == jg)).astype(
                jnp.bfloat16
            )
            xsend_vmem[k, :, :] = jnp.dot(
                oh, xb, preferred_element_type=jnp.float32
            ).astype(jnp.bfloat16)
            if k == 0:
                local_cp = pltpu.make_async_copy(
                    xsend_vmem.at[0], xrecv_ref.at[my], local_sem
                )
                local_cp.start()
            else:
                rdma = pltpu.make_async_remote_copy(
                    src_ref=xsend_vmem.at[k],
                    dst_ref=xrecv_ref.at[my],
                    send_sem=send1.at[k - 1],
                    recv_sem=recv1.at[k - 1],
                    device_id=(dest,),
                    device_id_type=pl.DeviceIdType.MESH,
                )
                rdma.start()
                disp.append(rdma)
        local_cp.wait()
        for rdma in disp:
            rdma.wait_recv()

        def expert_step(s, carry):
            cw = pltpu.make_async_copy(ew_ref.at[s], w_vmem, w_sem)
            cw.start()
            cx = pltpu.make_async_copy(
                xrecv_ref.at[:, pl.ds(s * CAP, CAP)], xa_vmem, xa_sem
            )
            cx.start()
            cx.wait()
            cw.wait()
            xa = xa_vmem[:, :, :].reshape(N_DEV * CAP, d_model)
            y = jnp.dot(xa, w_vmem[:, :], preferred_element_type=jnp.float32)
            ya_vmem[:, :, :] = y.astype(jnp.bfloat16).reshape(
                N_DEV, CAP, d_model
            )
            cy = pltpu.make_async_copy(
                ya_vmem, ysend_ref.at[:, pl.ds(s * CAP, CAP)], ya_sem
            )
            cy.start()
            cy.wait()
            return carry

        lax.fori_loop(0, N_LOCAL_E, expert_step, 0)

        ret = []
        for k in range(1, N_DEV):
            dest = lax.rem(my + k, N_DEV)
            rdma = pltpu.make_async_remote_copy(
                src_ref=ysend_ref.at[dest],
                dst_ref=yret_ref.at[my],
                send_sem=send2.at[k - 1],
                recv_sem=recv2.at[k - 1],
                device_id=(dest,),
                device_id_type=pl.DeviceIdType.MESH,
            )
            rdma.start()
            ret.append(rdma)
        local_cp = pltpu.make_async_copy(
            ysend_ref.at[my], yret_ref.at[my], local_sem
        )
        local_cp.start()

        out_ref[:, :] = jnp.zeros_like(out_ref)
        iota_jr = lax.broadcasted_iota(jnp.int32, (n_tok, NSLOT), 1)
        for k in range(N_DEV):
            o = lax.rem(my - k + N_DEV, N_DEV)
            if k == 0:
                local_cp.wait()
            else:
                ret[k - 1].wait_recv()
            cy = pltpu.make_async_copy(yret_ref.at[o], y_vmem, y_sem)
            cy.start()
            cy.wait()
            jg = iota_jr + o * NSLOT
            m = (
                jnp.where(s0c_ref[:, :] == jg, g0c_ref[:, :], 0.0)
                + jnp.where(s1c_ref[:, :] == jg, g1c_ref[:, :], 0.0)
            ).astype(jnp.bfloat16)
            out_ref[:, :] += jnp.dot(
                m, y_vmem[:, :], preferred_element_type=jnp.float32
            )
        for rdma in disp:
            rdma.wait_send()
        for rdma in ret:
            rdma.wait_send()

    return pl.pallas_call(
        body,
        out_shape=(
            jax.ShapeDtypeStruct((n_tok, d_model), jnp.float32),
            jax.ShapeDtypeStruct((N_DEV, NSLOT, d_model), jnp.bfloat16),
            jax.ShapeDtypeStruct((N_DEV, NSLOT, d_model), jnp.bfloat16),
            jax.ShapeDtypeStruct((N_DEV, NSLOT, d_model), jnp.bfloat16),
        ),
        in_specs=[
            pl.BlockSpec(memory_space=pltpu.MemorySpace.VMEM),
            pl.BlockSpec(memory_space=pltpu.MemorySpace.VMEM),
            pl.BlockSpec(memory_space=pltpu.MemorySpace.VMEM),
            pl.BlockSpec(memory_space=pltpu.MemorySpace.VMEM),
            pl.BlockSpec(memory_space=pltpu.MemorySpace.VMEM),
            pl.BlockSpec(memory_space=pltpu.MemorySpace.VMEM),
            pl.BlockSpec(memory_space=pltpu.MemorySpace.VMEM),
            pl.BlockSpec(memory_space=pl.ANY),
        ],
        out_specs=(
            pl.BlockSpec(memory_space=pltpu.MemorySpace.VMEM),
            pl.BlockSpec(memory_space=pl.ANY),
            pl.BlockSpec(memory_space=pl.ANY),
            pl.BlockSpec(memory_space=pl.ANY),
        ),
        scratch_shapes=[
            pltpu.VMEM((N_DEV, NSLOT, d_model), jnp.bfloat16),
            pltpu.VMEM((N_DEV, CAP, d_model), jnp.bfloat16),
            pltpu.VMEM((d_model, d_model), jnp.bfloat16),
            pltpu.VMEM((N_DEV, CAP, d_model), jnp.bfloat16),
            pltpu.VMEM((NSLOT, d_model), jnp.bfloat16),
            pltpu.SemaphoreType.DMA,
            pltpu.SemaphoreType.DMA,
            pltpu.SemaphoreType.DMA,
            pltpu.SemaphoreType.DMA,
            pltpu.SemaphoreType.DMA,
            pltpu.SemaphoreType.DMA((N_DEV - 1,)),
            pltpu.SemaphoreType.DMA((N_DEV - 1,)),
            pltpu.SemaphoreType.DMA((N_DEV - 1,)),
            pltpu.SemaphoreType.DMA((N_DEV - 1,)),
        ],
        compiler_params=pltpu.CompilerParams(
            has_side_effects=True,
            vmem_limit_bytes=120 * 1024 * 1024,
        ),
    )


def kernel(x, router_W, route_idx, expert_W):
    n_tok, d_model = x.shape
    n_exp = router_W.shape[1]

    scores = jnp.dot(x, router_W, preferred_element_type=jnp.float32)
    p = jax.nn.softmax(scores, axis=-1)
    iota_e = jnp.arange(n_exp, dtype=jnp.int32)
    e0 = route_idx[:, 0:1]
    e1 = route_idx[:, 1:2]
    g0 = jnp.sum(jnp.where(e0 == iota_e[None, :], p, 0.0), axis=1)
    g1 = jnp.sum(jnp.where(e1 == iota_e[None, :], p, 0.0), axis=1)
    gs = g0 + g1
    g0 = g0 / gs
    g1 = g1 / gs

    flat_e = route_idx.reshape(-1)
    onehot = flat_e[:, None] == iota_e[None, :]
    cum = jnp.cumsum(onehot.astype(jnp.int32), axis=0)
    pos = jnp.sum(jnp.where(onehot, cum - 1, 0), axis=1)
    slot = jnp.where(pos < CAP, flat_e * CAP + pos, n_exp * CAP + 1)
    slot2 = slot.reshape(n_tok, 2)
    s0 = slot2[:, 0]
    s1 = slot2[:, 1]

    out, _, _, _ = _a2a_kernel(n_tok, d_model)(
        x,
        s0.reshape(1, -1),
        s1.reshape(1, -1),
        s0.reshape(-1, 1),
        s1.reshape(-1, 1),
        g0.reshape(-1, 1),
        g1.reshape(-1, 1),
        expert_W.astype(jnp.bfloat16),
    )
    return out


# device time: 376901 ns/iter; 1.8951x vs baseline; 1.0385x over previous
import jax
import jax.numpy as jnp
from jax import lax
from jax.experimental import pallas as pl
from jax.experimental.pallas import tpu as pltpu

N_DEV = 8
N_LOCAL_E = 8
CAP = 128
NSLOT = N_LOCAL_E * CAP


def _a2a_kernel(n_tok, d_model):
    def body(x_ref, s0r_ref, s1r_ref, s0c_ref, s1c_ref, g0c_ref, g1c_ref,
             ew_ref,
             out_ref, xrecv_ref, ysend_ref, yret_ref,
             xsend_vmem, xa_vmem, w_vmem, ya_vmem, y_vmem,
             local_sem, xa_sem, w_sem, ya_sem, y_sem,
             send1, recv1, send2, recv2):
        my = lax.axis_index("i")

        xb = x_ref[:, :].astype(jnp.bfloat16)

        iota_jc = lax.broadcasted_iota(jnp.int32, (NSLOT, n_tok), 0)
        disp = []
        local_cp = None
        for k in range(N_DEV):
            dest = lax.rem(my + k, N_DEV)
            jg = iota_jc + dest * NSLOT
            oh = ((s0r_ref[:, :] == jg) | (s1r_ref[:, :] == jg)).astype(
                jnp.bfloat16
            )
            xsend_vmem[k, :, :] = jnp.dot(
                oh, xb, preferred_element_type=jnp.float32
            ).astype(jnp.bfloat16)
            if k == 0:
                local_cp = pltpu.make_async_copy(
                    xsend_vmem.at[0], xrecv_ref.at[my], local_sem
                )
                local_cp.start()
            else:
                rdma = pltpu.make_async_remote_copy(
                    src_ref=xsend_vmem.at[k],
                    dst_ref=xrecv_ref.at[my],
                    send_sem=send1.at[k - 1],
                    recv_sem=recv1.at[k - 1],
                    device_id=(dest,),
                    device_id_type=pl.DeviceIdType.MESH,
                )
                rdma.start()
                disp.append(rdma)
        local_cp.wait()
        for rdma in disp:
            rdma.wait_recv()

        def w_copy(s, b):
            return pltpu.make_async_copy(ew_ref.at[s], w_vmem.at[b], w_sem.at[b])

        def xa_copy(s, b):
            return pltpu.make_async_copy(
                xrecv_ref.at[:, pl.ds(s * CAP, CAP)], xa_vmem.at[b], xa_sem.at[b]
            )

        def ya_copy(s, b):
            return pltpu.make_async_copy(
                ya_vmem.at[b], ysend_ref.at[:, pl.ds(s * CAP, CAP)], ya_sem.at[b]
            )

        w_copy(0, 0).start()
        xa_copy(0, 0).start()
        for s in range(N_LOCAL_E):
            b = s % 2
            if s + 1 < N_LOCAL_E:
                w_copy(s + 1, 1 - b).start()
                xa_copy(s + 1, 1 - b).start()
            w_copy(s, b).wait()
            xa_copy(s, b).wait()
            if s >= 2:
                ya_copy(s - 2, b).wait()
            xa = xa_vmem[b].reshape(N_DEV * CAP, d_model)
            y = jnp.dot(xa, w_vmem[b], preferred_element_type=jnp.float32)
            ya_vmem[b, :, :, :] = y.astype(jnp.bfloat16).reshape(
                N_DEV, CAP, d_model
            )
            ya_copy(s, b).start()
        ya_copy(N_LOCAL_E - 2, 0).wait()
        ya_copy(N_LOCAL_E - 1, 1).wait()

        ret = []
        for k in range(1, N_DEV):
            dest = lax.rem(my + k, N_DEV)
            rdma = pltpu.make_async_remote_copy(
                src_ref=ysend_ref.at[dest],
                dst_ref=yret_ref.at[my],
                send_sem=send2.at[k - 1],
                recv_sem=recv2.at[k - 1],
                device_id=(dest,),
                device_id_type=pl.DeviceIdType.MESH,
            )
            rdma.start()
            ret.append(rdma)
        local_cp = pltpu.make_async_copy(
            ysend_ref.at[my], yret_ref.at[my], local_sem
        )
        local_cp.start()

        out_ref[:, :] = jnp.zeros_like(out_ref)
        iota_jr = lax.broadcasted_iota(jnp.int32, (n_tok, NSLOT), 1)
        for k in range(N_DEV):
            o = lax.rem(my - k + N_DEV, N_DEV)
            if k == 0:
                local_cp.wait()
            else:
                ret[k - 1].wait_recv()
            cy = pltpu.make_async_copy(yret_ref.at[o], y_vmem, y_sem)
            cy.start()
            jg = iota_jr + o * NSLOT
            m = (
                jnp.where(s0c_ref[:, :] == jg, g0c_ref[:, :], 0.0)
                + jnp.where(s1c_ref[:, :] == jg, g1c_ref[:, :], 0.0)
            ).astype(jnp.bfloat16)
            cy.wait()
            out_ref[:, :] += jnp.dot(
                m, y_vmem[:, :], preferred_element_type=jnp.float32
            )
        for rdma in disp:
            rdma.wait_send()
        for rdma in ret:
            rdma.wait_send()

    return pl.pallas_call(
        body,
        out_shape=(
            jax.ShapeDtypeStruct((n_tok, d_model), jnp.float32),
            jax.ShapeDtypeStruct((N_DEV, NSLOT, d_model), jnp.bfloat16),
            jax.ShapeDtypeStruct((N_DEV, NSLOT, d_model), jnp.bfloat16),
            jax.ShapeDtypeStruct((N_DEV, NSLOT, d_model), jnp.bfloat16),
        ),
        in_specs=[
            pl.BlockSpec(memory_space=pltpu.MemorySpace.VMEM),
            pl.BlockSpec(memory_space=pltpu.MemorySpace.VMEM),
            pl.BlockSpec(memory_space=pltpu.MemorySpace.VMEM),
            pl.BlockSpec(memory_space=pltpu.MemorySpace.VMEM),
            pl.BlockSpec(memory_space=pltpu.MemorySpace.VMEM),
            pl.BlockSpec(memory_space=pltpu.MemorySpace.VMEM),
            pl.BlockSpec(memory_space=pltpu.MemorySpace.VMEM),
            pl.BlockSpec(memory_space=pl.ANY),
        ],
        out_specs=(
            pl.BlockSpec(memory_space=pltpu.MemorySpace.VMEM),
            pl.BlockSpec(memory_space=pl.ANY),
            pl.BlockSpec(memory_space=pl.ANY),
            pl.BlockSpec(memory_space=pl.ANY),
        ),
        scratch_shapes=[
            pltpu.VMEM((N_DEV, NSLOT, d_model), jnp.bfloat16),
            pltpu.VMEM((2, N_DEV, CAP, d_model), jnp.bfloat16),
            pltpu.VMEM((2, d_model, d_model), jnp.bfloat16),
            pltpu.VMEM((2, N_DEV, CAP, d_model), jnp.bfloat16),
            pltpu.VMEM((NSLOT, d_model), jnp.bfloat16),
            pltpu.SemaphoreType.DMA,
            pltpu.SemaphoreType.DMA((2,)),
            pltpu.SemaphoreType.DMA((2,)),
            pltpu.SemaphoreType.DMA((2,)),
            pltpu.SemaphoreType.DMA,
            pltpu.SemaphoreType.DMA((N_DEV - 1,)),
            pltpu.SemaphoreType.DMA((N_DEV - 1,)),
            pltpu.SemaphoreType.DMA((N_DEV - 1,)),
            pltpu.SemaphoreType.DMA((N_DEV - 1,)),
        ],
        compiler_params=pltpu.CompilerParams(
            has_side_effects=True,
            vmem_limit_bytes=120 * 1024 * 1024,
        ),
    )


def kernel(x, router_W, route_idx, expert_W):
    n_tok, d_model = x.shape
    n_exp = router_W.shape[1]

    scores = jnp.dot(x, router_W, preferred_element_type=jnp.float32)
    p = jax.nn.softmax(scores, axis=-1)
    iota_e = jnp.arange(n_exp, dtype=jnp.int32)
    e0 = route_idx[:, 0:1]
    e1 = route_idx[:, 1:2]
    g0 = jnp.sum(jnp.where(e0 == iota_e[None, :], p, 0.0), axis=1)
    g1 = jnp.sum(jnp.where(e1 == iota_e[None, :], p, 0.0), axis=1)
    gs = g0 + g1
    g0 = g0 / gs
    g1 = g1 / gs

    flat_e = route_idx.reshape(-1)
    onehot = flat_e[:, None] == iota_e[None, :]
    cum = jnp.cumsum(onehot.astype(jnp.int32), axis=0)
    pos = jnp.sum(jnp.where(onehot, cum - 1, 0), axis=1)
    slot = jnp.where(pos < CAP, flat_e * CAP + pos, n_exp * CAP + 1)
    slot2 = slot.reshape(n_tok, 2)
    s0 = slot2[:, 0]
    s1 = slot2[:, 1]

    out, _, _, _ = _a2a_kernel(n_tok, d_model)(
        x,
        s0.reshape(1, -1),
        s1.reshape(1, -1),
        s0.reshape(-1, 1),
        s1.reshape(-1, 1),
        g0.reshape(-1, 1),
        g1.reshape(-1, 1),
        expert_W.astype(jnp.bfloat16),
    )
    return out
